# baseline (speedup 1.0000x reference)
import functools

import jax
import jax.numpy as jnp
from jax import lax
from jax.experimental import pallas as pl
from jax.experimental.pallas import tpu as pltpu
from jax.experimental.pallas import tpu_sc as plsc

N = 10000
E = 320000
D = 128
T = 128
G = 64

NC = 2
NS = 16
NW = NC * NS
EW = E // NW
K = 80
NCHUNK = EW // K
NRA = 10240
RPT = NRA // NS

_mesh = plsc.VectorSubcoreMesh(core_axis_name="c", subcore_axis_name="s")


@functools.partial(
    pl.kernel,
    out_type=jax.ShapeDtypeStruct((NC, NRA, D), jnp.float32),
    mesh=_mesh,
    scratch_types=[
        pltpu.VMEM((EW,), jnp.int32),
        pltpu.VMEM((NCHUNK, K), jnp.int32),
        pltpu.VMEM((K, D), jnp.float32),
        pltpu.VMEM((K, D), jnp.float32),
        pltpu.VMEM_SHARED((NRA, D), jnp.float32),
        pltpu.SemaphoreType.DMA,
        pltpu.SemaphoreType.DMA,
        pltpu.SemaphoreType.DMA,
        pltpu.SemaphoreType.DMA,
        pltpu.SemaphoreType.DMA,
    ],
)
def _sc_aggregate(h_hbm, src_hbm, dst_hbm, zeros_hbm, out_hbm,
                  src_v, dst_v, rows_a, rows_b, agg_sh,
                  sem_a, sem_b, sem_z, sem_sa, sem_sb):
    cid = lax.axis_index("c")
    sid = lax.axis_index("s")
    wid = sid * NC + cid

    pltpu.async_copy(src_hbm.at[wid], src_v, sem_a)
    pltpu.async_copy(dst_hbm.at[wid], dst_v, sem_b)
    pltpu.async_copy(zeros_hbm, agg_sh.at[pl.ds(sid * RPT, RPT)], sem_z)
    pltpu.make_async_copy(src_hbm.at[wid], src_v, sem_a).wait()
    pltpu.make_async_copy(dst_hbm.at[wid], dst_v, sem_b).wait()

    pltpu.async_copy(h_hbm.at[src_v.at[pl.ds(0, K)]], rows_a, sem_a)
    pltpu.async_copy(h_hbm.at[src_v.at[pl.ds(K, K)]], rows_b, sem_b)
    pltpu.make_async_copy(zeros_hbm, agg_sh.at[pl.ds(sid * RPT, RPT)], sem_z).wait()
    plsc.subcore_barrier()

    def body(t, carry):
        i = 2 * t
        pltpu.make_async_copy(h_hbm.at[pl.ds(0, K)], rows_a, sem_a).wait()
        ca = pltpu.async_copy(rows_a, agg_sh.at[dst_v.at[i]], sem_sa, add=True)
        pltpu.make_async_copy(h_hbm.at[pl.ds(0, K)], rows_b, sem_b).wait()
        cb = pltpu.async_copy(rows_b, agg_sh.at[dst_v.at[i + 1]], sem_sb, add=True)
        ca.wait()
        pltpu.async_copy(h_hbm.at[src_v.at[pl.ds((i + 2) * K, K)]], rows_a, sem_a)
        cb.wait()
        pltpu.async_copy(h_hbm.at[src_v.at[pl.ds((i + 3) * K, K)]], rows_b, sem_b)
        return carry

    lax.fori_loop(0, (NCHUNK - 3) // 2, body, 0)

    pltpu.make_async_copy(h_hbm.at[pl.ds(0, K)], rows_a, sem_a).wait()
    ca = pltpu.async_copy(rows_a, agg_sh.at[dst_v.at[NCHUNK - 3]], sem_sa, add=True)
    pltpu.make_async_copy(h_hbm.at[pl.ds(0, K)], rows_b, sem_b).wait()
    cb = pltpu.async_copy(rows_b, agg_sh.at[dst_v.at[NCHUNK - 2]], sem_sb, add=True)
    ca.wait()
    pltpu.async_copy(h_hbm.at[src_v.at[pl.ds((NCHUNK - 1) * K, K)]], rows_a, sem_a)
    pltpu.make_async_copy(h_hbm.at[pl.ds(0, K)], rows_a, sem_a).wait()
    pltpu.sync_copy(rows_a, agg_sh.at[dst_v.at[NCHUNK - 1]], add=True)
    cb.wait()

    plsc.subcore_barrier()
    pltpu.sync_copy(agg_sh.at[pl.ds(sid * RPT, RPT)],
                    out_hbm.at[cid, pl.ds(sid * RPT, RPT)])


def _dot(a, b):
    return jnp.dot(a, b, preferred_element_type=jnp.float32,
                   precision=lax.Precision.HIGHEST)


def _tc_layer_body(p_ref, w_ref, b_ref, o_ref):
    acc = p_ref[0] + p_ref[1]
    o_ref[...] = jnp.maximum(_dot(acc, w_ref[...]) + b_ref[...], 0.0)


_RB = 2000


def _tc_layer(p, w, b2d):
    return pl.pallas_call(
        _tc_layer_body,
        grid=(N // _RB,),
        in_specs=[
            pl.BlockSpec((NC, _RB, D), lambda i: (0, i, 0)),
            pl.BlockSpec((D, D), lambda i: (0, 0)),
            pl.BlockSpec((1, D), lambda i: (0, 0)),
        ],
        out_specs=pl.BlockSpec((_RB, D), lambda i: (i, 0)),
        out_shape=jax.ShapeDtypeStruct((N, D), jnp.float32),
    )(p, w, b2d)


def _tc_head_body(p_ref, w3_ref, b3_ref, batch_ref, wout_ref, bout_ref, o_ref):
    acc = p_ref[0] + p_ref[1]
    h3 = jnp.maximum(_dot(acc, w3_ref[...]) + b3_ref[...], 0.0)
    gids = lax.broadcasted_iota(jnp.int32, (G, N), 0)
    onehot = (batch_ref[...] == gids).astype(jnp.float32)
    counts = jnp.sum(onehot, axis=1, keepdims=True)
    sums = _dot(onehot, h3)
    hg = sums / jnp.maximum(counts, 1.0)
    o_ref[...] = _dot(hg, wout_ref[...]) + bout_ref[...]


def _tc_head(p, w3, b3_2d, batch2d, wout, bout2d):
    return pl.pallas_call(
        _tc_head_body,
        grid=(1,),
        in_specs=[
            pl.BlockSpec((NC, N, D), lambda i: (0, 0, 0)),
            pl.BlockSpec((D, D), lambda i: (0, 0)),
            pl.BlockSpec((1, D), lambda i: (0, 0)),
            pl.BlockSpec((1, N), lambda i: (0, 0)),
            pl.BlockSpec((D, T), lambda i: (0, 0)),
            pl.BlockSpec((1, T), lambda i: (0, 0)),
        ],
        out_specs=pl.BlockSpec((G, T), lambda i: (0, 0)),
        out_shape=jax.ShapeDtypeStruct((G, T), jnp.float32),
    )(p, w3, b3_2d, batch2d, wout, bout2d)


def kernel(x, edge_index, batch, W1, b1, W2, b2, W3, b3, Wout, bout):
    src = edge_index[0].astype(jnp.int32).reshape(NW, EW)
    dst = edge_index[1].astype(jnp.int32).reshape(NW, NCHUNK, K)
    zeros = jnp.zeros((RPT, D), jnp.float32)
    batch2d = batch.astype(jnp.int32).reshape(1, N)

    h = x
    for (w, b) in ((W1, b1), (W2, b2)):
        p = _sc_aggregate(h, src, dst, zeros)
        h = _tc_layer(p, w, b.reshape(1, D))
    p = _sc_aggregate(h, src, dst, zeros)
    return _tc_head(p, W3, b3.reshape(1, D), batch2d, Wout, bout.reshape(1, D))


# speedup vs baseline: 1.3038x; 1.3038x over previous
import functools

import jax
import jax.numpy as jnp
from jax import lax
from jax.experimental import pallas as pl
from jax.experimental.pallas import tpu as pltpu
from jax.experimental.pallas import tpu_sc as plsc

N = 10000
E = 320000
D = 128
T = 128
G = 64

NC = 2
NS = 16
NW = NC * NS
EW = E // NW
K = 96
EWP = 10080
PAD = EWP - EW
NCHUNK = EWP // K
NRA = 10112
RPT = NRA // NS

_mesh = plsc.VectorSubcoreMesh(core_axis_name="c", subcore_axis_name="s")


@functools.partial(
    pl.kernel,
    out_type=jax.ShapeDtypeStruct((NC, NRA, D), jnp.float32),
    mesh=_mesh,
    scratch_types=[
        pltpu.VMEM((EWP,), jnp.int32),
        pltpu.VMEM((NCHUNK, K), jnp.int32),
        pltpu.VMEM((K, D), jnp.float32),
        pltpu.VMEM((K, D), jnp.float32),
        pltpu.VMEM_SHARED((NRA, D), jnp.float32),
        pltpu.SemaphoreType.DMA,
        pltpu.SemaphoreType.DMA,
        pltpu.SemaphoreType.DMA,
    ],
)
def _sc_aggregate(h_hbm, src_hbm, dst_hbm, zeros_hbm, out_hbm,
                  src_v, dst_v, rows_a, rows_b, agg_sh, sem_a, sem_b, sem_z):
    cid = lax.axis_index("c")
    sid = lax.axis_index("s")
    wid = sid * NC + cid

    pltpu.async_copy(src_hbm.at[wid], src_v, sem_a)
    pltpu.async_copy(dst_hbm.at[wid], dst_v, sem_b)
    pltpu.async_copy(zeros_hbm, agg_sh.at[pl.ds(sid * RPT, RPT)], sem_z)
    pltpu.make_async_copy(src_hbm.at[wid], src_v, sem_a).wait()
    pltpu.make_async_copy(dst_hbm.at[wid], dst_v, sem_b).wait()

    pltpu.async_copy(h_hbm.at[src_v.at[pl.ds(0, K)]], rows_a, sem_a)
    pltpu.async_copy(h_hbm.at[src_v.at[pl.ds(K, K)]], rows_b, sem_b)
    pltpu.make_async_copy(zeros_hbm, agg_sh.at[pl.ds(sid * RPT, RPT)], sem_z).wait()
    plsc.subcore_barrier()

    def body(t, carry):
        i = 2 * t
        pltpu.make_async_copy(h_hbm.at[pl.ds(0, K)], rows_a, sem_a).wait()
        pltpu.sync_copy(rows_a, agg_sh.at[dst_v.at[i]], add=True)
        pltpu.async_copy(h_hbm.at[src_v.at[pl.ds((i + 2) * K, K)]], rows_a, sem_a)
        pltpu.make_async_copy(h_hbm.at[pl.ds(0, K)], rows_b, sem_b).wait()
        pltpu.sync_copy(rows_b, agg_sh.at[dst_v.at[i + 1]], add=True)
        pltpu.async_copy(h_hbm.at[src_v.at[pl.ds((i + 3) * K, K)]], rows_b, sem_b)
        return carry

    lax.fori_loop(0, (NCHUNK - 3) // 2, body, 0)

    pltpu.make_async_copy(h_hbm.at[pl.ds(0, K)], rows_a, sem_a).wait()
    pltpu.sync_copy(rows_a, agg_sh.at[dst_v.at[NCHUNK - 3]], add=True)
    pltpu.async_copy(h_hbm.at[src_v.at[pl.ds((NCHUNK - 1) * K, K)]], rows_a, sem_a)
    pltpu.make_async_copy(h_hbm.at[pl.ds(0, K)], rows_b, sem_b).wait()
    pltpu.sync_copy(rows_b, agg_sh.at[dst_v.at[NCHUNK - 2]], add=True)
    pltpu.make_async_copy(h_hbm.at[pl.ds(0, K)], rows_a, sem_a).wait()
    pltpu.sync_copy(rows_a, agg_sh.at[dst_v.at[NCHUNK - 1]], add=True)

    plsc.subcore_barrier()
    pltpu.sync_copy(agg_sh.at[pl.ds(sid * RPT, RPT)],
                    out_hbm.at[cid, pl.ds(sid * RPT, RPT)])


def _dot(a, b):
    return jnp.dot(a, b, preferred_element_type=jnp.float32,
                   precision=lax.Precision.HIGHEST)


def _tc_layer_body(p_ref, w_ref, b_ref, o_ref):
    acc = p_ref[0] + p_ref[1]
    o_ref[...] = jnp.maximum(_dot(acc, w_ref[...]) + b_ref[...], 0.0)


_RB = 2000


def _tc_layer(p, w, b2d):
    return pl.pallas_call(
        _tc_layer_body,
        grid=(N // _RB,),
        in_specs=[
            pl.BlockSpec((NC, _RB, D), lambda i: (0, i, 0)),
            pl.BlockSpec((D, D), lambda i: (0, 0)),
            pl.BlockSpec((1, D), lambda i: (0, 0)),
        ],
        out_specs=pl.BlockSpec((_RB, D), lambda i: (i, 0)),
        out_shape=jax.ShapeDtypeStruct((N, D), jnp.float32),
    )(p, w, b2d)


def _tc_head_body(p_ref, w3_ref, b3_ref, batch_ref, wout_ref, bout_ref, o_ref):
    acc = p_ref[0] + p_ref[1]
    h3 = jnp.maximum(_dot(acc, w3_ref[...]) + b3_ref[...], 0.0)
    gids = lax.broadcasted_iota(jnp.int32, (G, N), 0)
    onehot = (batch_ref[...] == gids).astype(jnp.float32)
    counts = jnp.sum(onehot, axis=1, keepdims=True)
    sums = _dot(onehot, h3)
    hg = sums / jnp.maximum(counts, 1.0)
    o_ref[...] = _dot(hg, wout_ref[...]) + bout_ref[...]


def _tc_head(p, w3, b3_2d, batch2d, wout, bout2d):
    return pl.pallas_call(
        _tc_head_body,
        grid=(1,),
        in_specs=[
            pl.BlockSpec((NC, N, D), lambda i: (0, 0, 0)),
            pl.BlockSpec((D, D), lambda i: (0, 0)),
            pl.BlockSpec((1, D), lambda i: (0, 0)),
            pl.BlockSpec((1, N), lambda i: (0, 0)),
            pl.BlockSpec((D, T), lambda i: (0, 0)),
            pl.BlockSpec((1, T), lambda i: (0, 0)),
        ],
        out_specs=pl.BlockSpec((G, T), lambda i: (0, 0)),
        out_shape=jax.ShapeDtypeStruct((G, T), jnp.float32),
    )(p, w3, b3_2d, batch2d, wout, bout2d)


def kernel(x, edge_index, batch, W1, b1, W2, b2, W3, b3, Wout, bout):
    wi = jnp.arange(NW, dtype=jnp.int32)[:, None]
    ji = jnp.arange(PAD, dtype=jnp.int32)[None, :]
    pad_src = (wi * 997 + ji * 13) % N
    pad_dst = N + (wi * 37 + ji) % (NRA - N)
    src = jnp.concatenate(
        [edge_index[0].astype(jnp.int32).reshape(NW, EW), pad_src], axis=1)
    dst = jnp.concatenate(
        [edge_index[1].astype(jnp.int32).reshape(NW, EW), pad_dst], axis=1)
    dst = dst.reshape(NW, NCHUNK, K)
    zeros = jnp.zeros((RPT, D), jnp.float32)
    batch2d = batch.astype(jnp.int32).reshape(1, N)

    h = x
    for (w, b) in ((W1, b1), (W2, b2)):
        p = _sc_aggregate(h, src, dst, zeros)
        h = _tc_layer(p, w, b.reshape(1, D))
    p = _sc_aggregate(h, src, dst, zeros)
    return _tc_head(p, W3, b3.reshape(1, D), batch2d, Wout, bout.reshape(1, D))


# speedup vs baseline: 1.3174x; 1.0104x over previous
import functools

import jax
import jax.numpy as jnp
from jax import lax
from jax.experimental import pallas as pl
from jax.experimental.pallas import tpu as pltpu
from jax.experimental.pallas import tpu_sc as plsc

N = 10000
E = 320000
D = 128
T = 128
G = 64

NC = 2
NS = 16
NW = NC * NS
EW = E // NW
K = 104
EWP = 10088
PAD = EWP - EW
NCHUNK = EWP // K
NRA = 10112
RPT = NRA // NS

_mesh = plsc.VectorSubcoreMesh(core_axis_name="c", subcore_axis_name="s")


@functools.partial(
    pl.kernel,
    out_type=jax.ShapeDtypeStruct((NC, NRA, D), jnp.float32),
    mesh=_mesh,
    scratch_types=[
        pltpu.VMEM((EWP,), jnp.int32),
        pltpu.VMEM((NCHUNK, K), jnp.int32),
        pltpu.VMEM((K, D), jnp.float32),
        pltpu.VMEM((K, D), jnp.float32),
        pltpu.VMEM_SHARED((NRA, D), jnp.float32),
        pltpu.SemaphoreType.DMA,
        pltpu.SemaphoreType.DMA,
        pltpu.SemaphoreType.DMA,
    ],
)
def _sc_aggregate(h_hbm, src_hbm, dst_hbm, zeros_hbm, out_hbm,
                  src_v, dst_v, rows_a, rows_b, agg_sh, sem_a, sem_b, sem_z):
    cid = lax.axis_index("c")
    sid = lax.axis_index("s")
    wid = sid * NC + cid

    pltpu.async_copy(src_hbm.at[wid], src_v, sem_a)
    pltpu.async_copy(dst_hbm.at[wid], dst_v, sem_b)
    pltpu.async_copy(zeros_hbm, agg_sh.at[pl.ds(sid * RPT, RPT)], sem_z)
    pltpu.make_async_copy(src_hbm.at[wid], src_v, sem_a).wait()
    pltpu.make_async_copy(dst_hbm.at[wid], dst_v, sem_b).wait()

    pltpu.async_copy(h_hbm.at[src_v.at[pl.ds(0, K)]], rows_a, sem_a)
    pltpu.async_copy(h_hbm.at[src_v.at[pl.ds(K, K)]], rows_b, sem_b)
    pltpu.make_async_copy(zeros_hbm, agg_sh.at[pl.ds(sid * RPT, RPT)], sem_z).wait()
    plsc.subcore_barrier()

    def body(t, carry):
        i = 2 * t
        pltpu.make_async_copy(h_hbm.at[pl.ds(0, K)], rows_a, sem_a).wait()
        pltpu.sync_copy(rows_a, agg_sh.at[dst_v.at[i]], add=True)
        pltpu.async_copy(h_hbm.at[src_v.at[pl.ds((i + 2) * K, K)]], rows_a, sem_a)
        pltpu.make_async_copy(h_hbm.at[pl.ds(0, K)], rows_b, sem_b).wait()
        pltpu.sync_copy(rows_b, agg_sh.at[dst_v.at[i + 1]], add=True)
        pltpu.async_copy(h_hbm.at[src_v.at[pl.ds((i + 3) * K, K)]], rows_b, sem_b)
        return carry

    lax.fori_loop(0, (NCHUNK - 3) // 2, body, 0)

    pltpu.make_async_copy(h_hbm.at[pl.ds(0, K)], rows_a, sem_a).wait()
    pltpu.sync_copy(rows_a, agg_sh.at[dst_v.at[NCHUNK - 3]], add=True)
    pltpu.async_copy(h_hbm.at[src_v.at[pl.ds((NCHUNK - 1) * K, K)]], rows_a, sem_a)
    pltpu.make_async_copy(h_hbm.at[pl.ds(0, K)], rows_b, sem_b).wait()
    pltpu.sync_copy(rows_b, agg_sh.at[dst_v.at[NCHUNK - 2]], add=True)
    pltpu.make_async_copy(h_hbm.at[pl.ds(0, K)], rows_a, sem_a).wait()
    pltpu.sync_copy(rows_a, agg_sh.at[dst_v.at[NCHUNK - 1]], add=True)

    plsc.subcore_barrier()
    pltpu.sync_copy(agg_sh.at[pl.ds(sid * RPT, RPT)],
                    out_hbm.at[cid, pl.ds(sid * RPT, RPT)])


def _dot(a, b):
    return jnp.dot(a, b, preferred_element_type=jnp.float32,
                   precision=lax.Precision.HIGHEST)


def _tc_layer_body(p_ref, w_ref, b_ref, o_ref):
    acc = p_ref[0] + p_ref[1]
    o_ref[...] = jnp.maximum(_dot(acc, w_ref[...]) + b_ref[...], 0.0)


_RB = 2000


def _tc_layer(p, w, b2d):
    return pl.pallas_call(
        _tc_layer_body,
        grid=(N // _RB,),
        in_specs=[
            pl.BlockSpec((NC, _RB, D), lambda i: (0, i, 0)),
            pl.BlockSpec((D, D), lambda i: (0, 0)),
            pl.BlockSpec((1, D), lambda i: (0, 0)),
        ],
        out_specs=pl.BlockSpec((_RB, D), lambda i: (i, 0)),
        out_shape=jax.ShapeDtypeStruct((N, D), jnp.float32),
    )(p, w, b2d)


def _tc_head_body(p_ref, w3_ref, b3_ref, batch_ref, wout_ref, bout_ref, o_ref):
    acc = p_ref[0] + p_ref[1]
    h3 = jnp.maximum(_dot(acc, w3_ref[...]) + b3_ref[...], 0.0)
    gids = lax.broadcasted_iota(jnp.int32, (G, N), 0)
    onehot = (batch_ref[...] == gids).astype(jnp.float32)
    counts = jnp.sum(onehot, axis=1, keepdims=True)
    sums = _dot(onehot, h3)
    hg = sums / jnp.maximum(counts, 1.0)
    o_ref[...] = _dot(hg, wout_ref[...]) + bout_ref[...]


def _tc_head(p, w3, b3_2d, batch2d, wout, bout2d):
    return pl.pallas_call(
        _tc_head_body,
        grid=(1,),
        in_specs=[
            pl.BlockSpec((NC, N, D), lambda i: (0, 0, 0)),
            pl.BlockSpec((D, D), lambda i: (0, 0)),
            pl.BlockSpec((1, D), lambda i: (0, 0)),
            pl.BlockSpec((1, N), lambda i: (0, 0)),
            pl.BlockSpec((D, T), lambda i: (0, 0)),
            pl.BlockSpec((1, T), lambda i: (0, 0)),
        ],
        out_specs=pl.BlockSpec((G, T), lambda i: (0, 0)),
        out_shape=jax.ShapeDtypeStruct((G, T), jnp.float32),
    )(p, w3, b3_2d, batch2d, wout, bout2d)


def kernel(x, edge_index, batch, W1, b1, W2, b2, W3, b3, Wout, bout):
    wi = jnp.arange(NW, dtype=jnp.int32)[:, None]
    ji = jnp.arange(PAD, dtype=jnp.int32)[None, :]
    pad_src = (wi * 997 + ji * 13) % N
    pad_dst = N + (wi * 37 + ji) % (NRA - N)
    src = jnp.concatenate(
        [edge_index[0].astype(jnp.int32).reshape(NW, EW), pad_src], axis=1)
    dst = jnp.concatenate(
        [edge_index[1].astype(jnp.int32).reshape(NW, EW), pad_dst], axis=1)
    dst = dst.reshape(NW, NCHUNK, K)
    zeros = jnp.zeros((RPT, D), jnp.float32)
    batch2d = batch.astype(jnp.int32).reshape(1, N)

    h = x
    for (w, b) in ((W1, b1), (W2, b2)):
        p = _sc_aggregate(h, src, dst, zeros)
        h = _tc_layer(p, w, b.reshape(1, D))
    p = _sc_aggregate(h, src, dst, zeros)
    return _tc_head(p, W3, b3.reshape(1, D), batch2d, Wout, bout.reshape(1, D))
